# Initial kernel scaffold; baseline (speedup 1.0000x reference)
import jax
import jax.numpy as jnp
from jax.experimental import pallas as pl


def kernel(w1, b1, w2, b2, wf1, bf1, wf2, bf2, x_nchw):
    raise NotImplementedError("write your pallas kernel here")


# speedup vs baseline: 218.1735x; 218.1735x over previous
import numpy as np
import jax
import jax.numpy as jnp
from jax.experimental import pallas as pl
from jax.experimental.pallas import tpu as pltpu

_BB = 256


def _t1_indices():
    I, K, L, T, C = [], [], [], [], []
    for i in range(5):
        for j in range(5):
            for w in range(28):
                w_in = w + j - 2
                if not 0 <= w_in < 28:
                    continue
                for c in range(10):
                    I.append(i)
                    K.append(w_in)
                    L.append((w % 2) * 256 + (w // 2) * 10 + c)
                    T.append(i * 5 + j)
                    C.append(c)
    return tuple(np.asarray(a, np.int32) for a in (I, K, L, T, C))


def _t2_indices():
    I, K, L, T, CI, CO = [], [], [], [], [], []
    for i in range(5):
        for j in range(5):
            for w in range(14):
                w_in = w + j - 2
                if not 0 <= w_in < 14:
                    continue
                for ci in range(10):
                    for co in range(20):
                        I.append(i)
                        K.append(w_in * 10 + ci)
                        L.append((w % 2) * 256 + (w // 2) * 20 + co)
                        T.append(i * 5 + j)
                        CI.append(ci)
                        CO.append(co)
    return tuple(np.asarray(a, np.int32) for a in (I, K, L, T, CI, CO))


def _b1_indices():
    L, C = [], []
    for w in range(28):
        for c in range(10):
            L.append((w % 2) * 256 + (w // 2) * 10 + c)
            C.append(c)
    return np.asarray(L, np.int32), np.asarray(C, np.int32)


def _b2_indices():
    L, C = [], []
    for w in range(14):
        for c in range(20):
            L.append((w % 2) * 256 + (w // 2) * 20 + c)
            C.append(c)
    return np.asarray(L, np.int32), np.asarray(C, np.int32)


_T1_IDX = _t1_indices()
_T2_IDX = _t2_indices()
_B1_IDX = _b1_indices()
_B2_IDX = _b2_indices()


def _pack_weights(w1, b1, w2, b2, wf1, bf1):
    f32 = jnp.float32
    i1, k1, l1, t1, c1 = _T1_IDX
    T1 = jnp.zeros((5, 32, 512), f32).at[i1, k1, l1].set(w1[t1, c1])
    i2, k2, l2, t2, ci2, co2 = _T2_IDX
    T2 = jnp.zeros((5, 256, 512), f32).at[i2, k2, l2].set(w2[t2, ci2, co2])
    lb1, cb1 = _B1_IDX
    b1p = jnp.zeros((1, 512), f32).at[0, lb1].set(b1[0, cb1])
    lb2, cb2 = _B2_IDX
    b2p = jnp.zeros((1, 512), f32).at[0, lb2].set(b2[0, cb2])
    F1 = jnp.pad(wf1.reshape(7, 140, 128), ((0, 0), (0, 116), (0, 0)))
    bf16 = jnp.bfloat16
    return T1.astype(bf16), b1p, T2.astype(bf16), b2p, F1.astype(bf16)


def _lenet_kernel(x_ref, t1_ref, b1_ref, t2_ref, b2_ref, f1_ref, bf1_ref,
                  wf2_ref, bf2_ref, o_ref, a1_s):
    f32 = jnp.float32
    for h2 in range(14):
        pooled_w = []
        for hp in range(2):
            h = 2 * h2 + hp
            acc = None
            for i in range(5):
                h_in = h + i - 2
                if 0 <= h_in < 28:
                    d = jnp.dot(x_ref[h_in], t1_ref[i],
                                preferred_element_type=f32)
                    acc = d if acc is None else acc + d
            y = jnp.maximum(acc + b1_ref[...], 0.0)
            pooled_w.append(jnp.maximum(y[:, :256], y[:, 256:]))
        a1_s[h2] = jnp.maximum(pooled_w[0], pooled_w[1]).astype(jnp.bfloat16)

    accf = None
    for h2o in range(7):
        pooled_w = []
        for hp in range(2):
            h = 2 * h2o + hp
            acc = None
            for i in range(5):
                h_in = h + i - 2
                if 0 <= h_in < 14:
                    d = jnp.dot(a1_s[h_in], t2_ref[i],
                                preferred_element_type=f32)
                    acc = d if acc is None else acc + d
            y = jnp.maximum(acc + b2_ref[...], 0.0)
            pooled_w.append(jnp.maximum(y[:, :256], y[:, 256:]))
        feat = jnp.maximum(pooled_w[0], pooled_w[1]).astype(jnp.bfloat16)
        d = jnp.dot(feat, f1_ref[h2o], preferred_element_type=f32)
        accf = d if accf is None else accf + d

    hid = (accf + bf1_ref[...]).astype(jnp.bfloat16)
    out = jnp.dot(hid, wf2_ref[...], preferred_element_type=f32)
    o_ref[...] = out + bf2_ref[...]


def kernel(w1, b1, w2, b2, wf1, bf1, wf2, bf2, x_nchw):
    B = x_nchw.shape[0]
    bf16 = jnp.bfloat16
    T1, b1p, T2, b2p, F1 = _pack_weights(w1, b1, w2, b2, wf1, bf1)
    xb = jnp.pad(x_nchw[:, 0], ((0, 0), (0, 0), (0, 4)))
    xb = jnp.transpose(xb, (1, 0, 2)).astype(bf16)

    out = pl.pallas_call(
        _lenet_kernel,
        out_shape=jax.ShapeDtypeStruct((B, 128), jnp.float32),
        grid=(B // _BB,),
        in_specs=[
            pl.BlockSpec((28, _BB, 32), lambda i: (0, i, 0)),
            pl.BlockSpec((5, 32, 512), lambda i: (0, 0, 0)),
            pl.BlockSpec((1, 512), lambda i: (0, 0)),
            pl.BlockSpec((5, 256, 512), lambda i: (0, 0, 0)),
            pl.BlockSpec((1, 512), lambda i: (0, 0)),
            pl.BlockSpec((7, 256, 128), lambda i: (0, 0, 0)),
            pl.BlockSpec((1, 128), lambda i: (0, 0)),
            pl.BlockSpec((128, 128), lambda i: (0, 0)),
            pl.BlockSpec((1, 128), lambda i: (0, 0)),
        ],
        out_specs=pl.BlockSpec((_BB, 128), lambda i: (i, 0)),
        scratch_shapes=[pltpu.VMEM((14, _BB, 256), bf16)],
        compiler_params=pltpu.CompilerParams(
            dimension_semantics=("parallel",)),
    )(xb, T1, b1p, T2, b2p, F1, bf1, wf2.astype(bf16), bf2)
    return out[:, :10]


# speedup vs baseline: 248.7262x; 1.1400x over previous
import numpy as np
import jax
import jax.numpy as jnp
from jax.experimental import pallas as pl
from jax.experimental.pallas import tpu as pltpu

_BB = 256


def _t1_indices():
    I, K, L, T, C = [], [], [], [], []
    for i in range(5):
        for j in range(5):
            for w in range(28):
                w_in = w + j - 2
                if not 0 <= w_in < 28:
                    continue
                for c in range(10):
                    I.append(i)
                    K.append(w_in)
                    L.append((w % 2) * 256 + (w // 2) * 10 + c)
                    T.append(i * 5 + j)
                    C.append(c)
    return tuple(np.asarray(a, np.int32) for a in (I, K, L, T, C))


def _t2_indices():
    I, K, L, T, CI, CO = [], [], [], [], [], []
    for i in range(5):
        for j in range(5):
            for w in range(14):
                w_in = w + j - 2
                if not 0 <= w_in < 14:
                    continue
                for ci in range(10):
                    for co in range(20):
                        I.append(i)
                        K.append(w_in * 10 + ci)
                        L.append((w % 2) * 256 + (w // 2) * 20 + co)
                        T.append(i * 5 + j)
                        CI.append(ci)
                        CO.append(co)
    return tuple(np.asarray(a, np.int32) for a in (I, K, L, T, CI, CO))


def _b1_indices():
    L, C = [], []
    for w in range(28):
        for c in range(10):
            L.append((w % 2) * 256 + (w // 2) * 10 + c)
            C.append(c)
    return np.asarray(L, np.int32), np.asarray(C, np.int32)


def _b2_indices():
    L, C = [], []
    for w in range(14):
        for c in range(20):
            L.append((w % 2) * 256 + (w // 2) * 20 + c)
            C.append(c)
    return np.asarray(L, np.int32), np.asarray(C, np.int32)


_T1_IDX = _t1_indices()
_T2_IDX = _t2_indices()
_B1_IDX = _b1_indices()
_B2_IDX = _b2_indices()


def _pack_weights(w1, b1, w2, b2, wf1, bf1):
    f32 = jnp.float32
    i1, k1, l1, t1, c1 = _T1_IDX
    T1 = jnp.zeros((5, 28, 512), f32).at[i1, k1, l1].set(w1[t1, c1])
    i2, k2, l2, t2, ci2, co2 = _T2_IDX
    T2 = jnp.zeros((5, 256, 512), f32).at[i2, k2, l2].set(w2[t2, ci2, co2])
    lb1, cb1 = _B1_IDX
    b1p = jnp.zeros((1, 512), f32).at[0, lb1].set(b1[0, cb1])
    lb2, cb2 = _B2_IDX
    b2p = jnp.zeros((1, 512), f32).at[0, lb2].set(b2[0, cb2])
    F1 = jnp.pad(wf1.reshape(7, 140, 128), ((0, 0), (0, 116), (0, 0)))
    bf16 = jnp.bfloat16
    return T1.astype(bf16), b1p, T2.astype(bf16), b2p, F1.astype(bf16)


def _lenet_kernel(x_ref, t1_ref, b1_ref, t2_ref, b2_ref, f1_ref, bf1_ref,
                  wf2_ref, bf2_ref, o_ref, a1_s):
    f32 = jnp.float32
    xrow = [x_ref[:, h, :].astype(jnp.bfloat16) for h in range(28)]
    for h2 in range(14):
        pooled_w = []
        for hp in range(2):
            h = 2 * h2 + hp
            acc = None
            for i in range(5):
                h_in = h + i - 2
                if 0 <= h_in < 28:
                    d = jnp.dot(xrow[h_in], t1_ref[i],
                                preferred_element_type=f32)
                    acc = d if acc is None else acc + d
            y = jnp.maximum(acc + b1_ref[...], 0.0)
            pooled_w.append(jnp.maximum(y[:, :256], y[:, 256:]))
        a1_s[h2] = jnp.maximum(pooled_w[0], pooled_w[1]).astype(jnp.bfloat16)

    accf = None
    for h2o in range(7):
        pooled_w = []
        for hp in range(2):
            h = 2 * h2o + hp
            acc = None
            for i in range(5):
                h_in = h + i - 2
                if 0 <= h_in < 14:
                    d = jnp.dot(a1_s[h_in], t2_ref[i],
                                preferred_element_type=f32)
                    acc = d if acc is None else acc + d
            y = jnp.maximum(acc + b2_ref[...], 0.0)
            pooled_w.append(jnp.maximum(y[:, :256], y[:, 256:]))
        feat = jnp.maximum(pooled_w[0], pooled_w[1]).astype(jnp.bfloat16)
        d = jnp.dot(feat, f1_ref[h2o], preferred_element_type=f32)
        accf = d if accf is None else accf + d

    hid = (accf + bf1_ref[...]).astype(jnp.bfloat16)
    out = jnp.dot(hid, wf2_ref[...], preferred_element_type=f32)
    o_ref[...] = out + bf2_ref[...]


def kernel(w1, b1, w2, b2, wf1, bf1, wf2, bf2, x_nchw):
    B = x_nchw.shape[0]
    bf16 = jnp.bfloat16
    T1, b1p, T2, b2p, F1 = _pack_weights(w1, b1, w2, b2, wf1, bf1)
    xb = x_nchw.reshape(B, 28, 28)

    out = pl.pallas_call(
        _lenet_kernel,
        out_shape=jax.ShapeDtypeStruct((B, 128), jnp.float32),
        grid=(B // _BB,),
        in_specs=[
            pl.BlockSpec((_BB, 28, 28), lambda i: (i, 0, 0)),
            pl.BlockSpec((5, 28, 512), lambda i: (0, 0, 0)),
            pl.BlockSpec((1, 512), lambda i: (0, 0)),
            pl.BlockSpec((5, 256, 512), lambda i: (0, 0, 0)),
            pl.BlockSpec((1, 512), lambda i: (0, 0)),
            pl.BlockSpec((7, 256, 128), lambda i: (0, 0, 0)),
            pl.BlockSpec((1, 128), lambda i: (0, 0)),
            pl.BlockSpec((128, 128), lambda i: (0, 0)),
            pl.BlockSpec((1, 128), lambda i: (0, 0)),
        ],
        out_specs=pl.BlockSpec((_BB, 128), lambda i: (i, 0)),
        scratch_shapes=[pltpu.VMEM((14, _BB, 256), bf16)],
        compiler_params=pltpu.CompilerParams(
            dimension_semantics=("parallel",)),
    )(xb, T1, b1p, T2, b2p, F1, bf1, wf2.astype(bf16), bf2)
    return out[:, :10]


# speedup vs baseline: 371.8402x; 1.4950x over previous
import numpy as np
import jax
import jax.numpy as jnp
from jax.experimental import pallas as pl
from jax.experimental.pallas import tpu as pltpu

_BB = 256

_D1 = np.zeros((5, 28, 28), np.float32)
_D2 = np.zeros((5, 14, 14), np.float32)
for _j in range(5):
    for _w in range(28):
        _v = _w + _j - 2
        if 0 <= _v < 28:
            _D1[_j, _v, _w] = 1.0
for _j in range(5):
    for _w in range(14):
        _v = _w + _j - 2
        if 0 <= _v < 14:
            _D2[_j, _v, _w] = 1.0


def _pack_weights(w1, b1, w2, b2, wf1, bf1):
    bf16 = jnp.bfloat16
    d1 = jnp.asarray(_D1)
    d2 = jnp.asarray(_D2)
    w1r = w1[:, :10].reshape(5, 5, 10)
    t1 = jnp.einsum('jvw,ijc->ivwc', d1, w1r)
    t1 = t1.reshape(5, 28, 14, 2, 10).transpose(0, 1, 3, 2, 4)
    t1 = jnp.pad(t1.reshape(5, 28, 2, 140), ((0, 0), (0, 0), (0, 0), (0, 116)))
    T1 = jnp.pad(t1.reshape(5, 28, 512), ((0, 0), (0, 4), (0, 0)))
    w2r = w2[:, :10, :20].reshape(5, 5, 10, 20)
    t2 = jnp.einsum('jvw,ijcd->ivcwd', d2, w2r)
    t2 = t2.reshape(5, 140, 7, 2, 20).transpose(0, 1, 3, 2, 4)
    t2 = jnp.pad(t2.reshape(5, 140, 2, 140), ((0, 0), (0, 0), (0, 0), (0, 116)))
    T2 = jnp.pad(t2.reshape(5, 140, 512), ((0, 0), (0, 116), (0, 0)))
    b1p = jnp.pad(jnp.tile(b1[0, :10], 14), (0, 116))
    b1p = jnp.tile(b1p, 2).reshape(1, 512)
    b2p = jnp.pad(jnp.tile(b2[0, :20], 7), (0, 116))
    b2p = jnp.tile(b2p, 2).reshape(1, 512)
    F1 = jnp.pad(wf1.reshape(7, 140, 128), ((0, 0), (0, 116), (0, 0)))
    return T1.astype(bf16), b1p, T2.astype(bf16), b2p, F1.astype(bf16)


def _lenet_kernel(x_ref, t1_ref, b1_ref, t2_ref, b2_ref, f1_ref, bf1_ref,
                  wf2_ref, bf2_ref, o_ref, a1_s):
    f32 = jnp.float32
    xrow = [x_ref[:, 32 * h:32 * h + 32].astype(jnp.bfloat16)
            for h in range(28)]
    for h2 in range(14):
        pooled_w = []
        for hp in range(2):
            h = 2 * h2 + hp
            acc = None
            for i in range(5):
                h_in = h + i - 2
                if 0 <= h_in < 28:
                    d = jnp.dot(xrow[h_in], t1_ref[i],
                                preferred_element_type=f32)
                    acc = d if acc is None else acc + d
            y = jnp.maximum(acc + b1_ref[...], 0.0)
            pooled_w.append(jnp.maximum(y[:, :256], y[:, 256:]))
        a1_s[h2] = jnp.maximum(pooled_w[0], pooled_w[1]).astype(jnp.bfloat16)

    accf = None
    for h2o in range(7):
        pooled_w = []
        for hp in range(2):
            h = 2 * h2o + hp
            acc = None
            for i in range(5):
                h_in = h + i - 2
                if 0 <= h_in < 14:
                    d = jnp.dot(a1_s[h_in], t2_ref[i],
                                preferred_element_type=f32)
                    acc = d if acc is None else acc + d
            y = jnp.maximum(acc + b2_ref[...], 0.0)
            pooled_w.append(jnp.maximum(y[:, :256], y[:, 256:]))
        feat = jnp.maximum(pooled_w[0], pooled_w[1]).astype(jnp.bfloat16)
        d = jnp.dot(feat, f1_ref[h2o], preferred_element_type=f32)
        accf = d if accf is None else accf + d

    hid = (accf + bf1_ref[...]).astype(jnp.bfloat16)
    out = jnp.dot(hid, wf2_ref[...], preferred_element_type=f32)
    o_ref[...] = out + bf2_ref[...]


def kernel(w1, b1, w2, b2, wf1, bf1, wf2, bf2, x_nchw):
    B = x_nchw.shape[0]
    bf16 = jnp.bfloat16
    T1, b1p, T2, b2p, F1 = _pack_weights(w1, b1, w2, b2, wf1, bf1)
    xb = jnp.pad(x_nchw.reshape(B, 28, 28),
                 ((0, 0), (0, 0), (0, 4))).reshape(B, 896)

    out = pl.pallas_call(
        _lenet_kernel,
        out_shape=jax.ShapeDtypeStruct((B, 128), jnp.float32),
        grid=(B // _BB,),
        in_specs=[
            pl.BlockSpec((_BB, 896), lambda i: (i, 0)),
            pl.BlockSpec((5, 32, 512), lambda i: (0, 0, 0)),
            pl.BlockSpec((1, 512), lambda i: (0, 0)),
            pl.BlockSpec((5, 256, 512), lambda i: (0, 0, 0)),
            pl.BlockSpec((1, 512), lambda i: (0, 0)),
            pl.BlockSpec((7, 256, 128), lambda i: (0, 0, 0)),
            pl.BlockSpec((1, 128), lambda i: (0, 0)),
            pl.BlockSpec((128, 128), lambda i: (0, 0)),
            pl.BlockSpec((1, 128), lambda i: (0, 0)),
        ],
        out_specs=pl.BlockSpec((_BB, 128), lambda i: (i, 0)),
        scratch_shapes=[pltpu.VMEM((14, _BB, 256), bf16)],
        compiler_params=pltpu.CompilerParams(
            dimension_semantics=("parallel",)),
    )(xb, T1, b1p, T2, b2p, F1, bf1, wf2.astype(bf16), bf2)
    return out[:, :10]
